# baseline (speedup 1.0000x reference)
import jax
import jax.numpy as jnp
from jax import lax
from jax.experimental import pallas as pl
from jax.experimental.pallas import tpu as pltpu

_B, _N, _H = 8, 128, 768
_NEG_INF = float("-inf")


def _merge_body(set1_ref, set2_ref, out_ref, sim_ref, rm_ref, oh1_ref, oh2_ref):
    iota_col = lax.broadcasted_iota(jnp.int32, (1, _N), 1)
    iota_row = lax.broadcasted_iota(jnp.int32, (_N, 1), 0)

    for b in range(_B):
        x1 = set1_ref[b]
        x2 = set2_ref[b]
        n1 = jnp.sqrt(jnp.sum(x1 * x1, axis=-1, keepdims=True))
        n2 = jnp.sqrt(jnp.sum(x2 * x2, axis=-1, keepdims=True))
        s1 = x1 / jnp.maximum(n1, 1e-12)
        s2 = x2 / jnp.maximum(n2, 1e-12)
        sim_b = lax.dot_general(s1, s2, (((1,), (1,)), ((), ())),
                                preferred_element_type=jnp.float32)
        sim_ref[b] = sim_b
        rm_ref[b] = jnp.max(sim_b, axis=1, keepdims=True)

    def step(r, carry):
        for b in range(_B):
            rm = rm_ref[b]
            m = jnp.max(rm)
            i_b = jnp.min(jnp.where(rm == m, iota_row, _N * _N))
            row = sim_ref[b, pl.ds(i_b, 1), :]
            j_b = jnp.min(jnp.where(row == m, iota_col, _N * _N))
            new_row = jnp.where(iota_col == j_b, _NEG_INF, row)
            sim_ref[b, pl.ds(i_b, 1), :] = new_row
            rm_ref[b, pl.ds(i_b, 1), :] = jnp.max(new_row).reshape(1, 1)
            oh1_ref[b, pl.ds(r, 1), :] = jnp.where(
                iota_col == i_b, jnp.float32(0.5), jnp.float32(0.0))
            oh2_ref[b, pl.ds(r, 1), :] = jnp.where(
                iota_col == j_b, jnp.float32(0.5), jnp.float32(0.0))
        return carry

    lax.fori_loop(0, _N, step, 0)

    for b in range(_B):
        g1 = lax.dot_general(oh1_ref[b], set1_ref[b], (((1,), (0,)), ((), ())),
                             precision=lax.Precision.HIGHEST,
                             preferred_element_type=jnp.float32)
        g2 = lax.dot_general(oh2_ref[b], set2_ref[b], (((1,), (0,)), ((), ())),
                             precision=lax.Precision.HIGHEST,
                             preferred_element_type=jnp.float32)
        out_ref[b] = g1 + g2


def _merged_tokens(set1, set2):
    return pl.pallas_call(
        _merge_body,
        out_shape=jax.ShapeDtypeStruct((_B, _N, _H), jnp.float32),
        scratch_shapes=[
            pltpu.VMEM((_B, _N, _N), jnp.float32),
            pltpu.VMEM((_B, _N, 1), jnp.float32),
            pltpu.VMEM((_B, _N, _N), jnp.float32),
            pltpu.VMEM((_B, _N, _N), jnp.float32),
        ],
    )(set1, set2)


def kernel(K):
    batch, num_tokens, hidden = K.shape
    Kr = K.reshape(batch, num_tokens // 2, 2, hidden)
    set1 = Kr[:, :, 0, :]
    set2 = Kr[:, :, 1, :]
    merged = _merged_tokens(set1, set2)
    return (merged, num_tokens // 2)


# speedup vs baseline: 3.4874x; 3.4874x over previous
import jax
import jax.numpy as jnp
from jax import lax
from jax.experimental import pallas as pl
from jax.experimental.pallas import tpu as pltpu

_B, _N, _H = 8, 128, 768
_NEG_INF = float("-inf")
_BIG = 1 << 30


def _merge_body(set1_ref, set2_ref, out_ref, sim_ref):
    lane_iota = lax.broadcasted_iota(jnp.int32, (_N, _N), 1)
    sub_iota = lax.broadcasted_iota(jnp.int32, (_N, _N), 0)
    ij_iota = sub_iota * _N + lane_iota
    b_iota = lax.broadcasted_iota(jnp.int32, (_B, _N), 0)
    r_iota = lax.broadcasted_iota(jnp.int32, (_B, _N), 1)

    for b in range(_B):
        x1 = set1_ref[b]
        x2 = set2_ref[b]
        n1 = jnp.sqrt(jnp.sum(x1 * x1, axis=-1, keepdims=True))
        n2 = jnp.sqrt(jnp.sum(x2 * x2, axis=-1, keepdims=True))
        s1 = x1 / jnp.maximum(n1, 1e-12)
        s2 = x2 / jnp.maximum(n2, 1e-12)
        sim_ref[b] = lax.dot_general(s1, s2, (((1,), (1,)), ((), ())),
                                     preferred_element_type=jnp.float32)

    def step(r, ch):
        for b in range(_B):
            s = sim_ref[b]
            m = jnp.max(jnp.max(s, axis=1, keepdims=True), axis=0,
                        keepdims=True)
            cand = jnp.where(s == m, ij_iota, _BIG)
            chosen = jnp.min(jnp.min(cand, axis=1, keepdims=True), axis=0,
                             keepdims=True)
            sim_ref[b] = jnp.where(ij_iota == chosen, _NEG_INF, s)
            upd = (b_iota == b) & (r_iota == r)
            ch = jnp.where(upd, jnp.broadcast_to(chosen, (_B, _N)), ch)
        return ch

    ch = lax.fori_loop(0, _N, step, jnp.zeros((_B, _N), jnp.int32))
    i_idx = ch // _N
    j_idx = ch % _N

    for b in range(_B):
        oht1 = jnp.where(sub_iota == i_idx[b:b + 1, :], jnp.float32(0.5),
                         jnp.float32(0.0))
        oht2 = jnp.where(sub_iota == j_idx[b:b + 1, :], jnp.float32(0.5),
                         jnp.float32(0.0))
        g1 = lax.dot_general(oht1, set1_ref[b], (((0,), (0,)), ((), ())),
                             precision=lax.Precision.HIGHEST,
                             preferred_element_type=jnp.float32)
        g2 = lax.dot_general(oht2, set2_ref[b], (((0,), (0,)), ((), ())),
                             precision=lax.Precision.HIGHEST,
                             preferred_element_type=jnp.float32)
        out_ref[b] = g1 + g2


def _merged_tokens(set1, set2):
    return pl.pallas_call(
        _merge_body,
        out_shape=jax.ShapeDtypeStruct((_B, _N, _H), jnp.float32),
        scratch_shapes=[
            pltpu.VMEM((_B, _N, _N), jnp.float32),
        ],
    )(set1, set2)


def kernel(K):
    batch, num_tokens, hidden = K.shape
    Kr = K.reshape(batch, num_tokens // 2, 2, hidden)
    set1 = Kr[:, :, 0, :]
    set2 = Kr[:, :, 1, :]
    merged = _merged_tokens(set1, set2)
    return (merged, num_tokens // 2)
